# baseline (device time: 130089 ns/iter reference)
import jax
import jax.numpy as jnp
from jax import lax
from jax.experimental import pallas as pl
from jax.experimental.pallas import tpu as pltpu

N_DEV = 4
M_PER = 1024
M_HALF = M_PER // 2
K = 4096
N_PER = 2048
KT = 1024


def kernel(x, w_mat, scale_x, scale_w):
    my = lax.axis_index("i")
    scale = jnp.reshape(scale_x[0] * scale_w[0], (1, 1))

    def body(
        x_ref, w_hbm, s_ref, out_init, out_ref, comm_r, comm_l, w8_ref,
        wtile_ref, stage_ref, send_r, recv_r, send_l, recv_l, copy_sems,
        w_sem,
    ):
        del out_init
        my_pos = lax.axis_index("i")
        left = lax.rem(my_pos + N_DEV - 1, N_DEV)
        right = lax.rem(my_pos + 1, N_DEV)

        barrier = pltpu.get_barrier_semaphore()
        for nbr in (left, right):
            pl.semaphore_signal(
                barrier,
                inc=1,
                device_id=(nbr,),
                device_id_type=pl.DeviceIdType.MESH,
            )
        pl.semaphore_wait(barrier, 2)

        def hop(comm, sems_s, sems_r, tgt, h):
            return pltpu.make_async_remote_copy(
                src_ref=comm.at[h],
                dst_ref=comm.at[h + 1],
                send_sem=sems_s.at[h],
                recv_sem=sems_r.at[h],
                device_id=(tgt,),
                device_id_type=pl.DeviceIdType.MESH,
            )

        d_r = [hop(comm_r, send_r, recv_r, right, h) for h in range(N_DEV - 1)]
        d_l = [hop(comm_l, send_l, recv_l, left, h) for h in range(N_DEV - 1)]

        comm_r[0] = x_ref[:M_HALF, :].astype(jnp.float8_e4m3fn)
        d_r[0].start()
        comm_l[0] = x_ref[M_HALF:, :].astype(jnp.float8_e4m3fn)
        d_l[0].start()

        col0 = my_pos * N_PER
        for kt in range(K // KT):
            wcopy = pltpu.make_async_copy(
                w_hbm.at[pl.ds(kt * KT, KT), pl.ds(col0, N_PER)],
                wtile_ref,
                w_sem,
            )
            wcopy.start()
            wcopy.wait()
            w8_ref[pl.ds(kt * KT, KT), :] = wtile_ref[...].astype(
                jnp.float8_e5m2
            )

        pending = [None, None]
        n_emitted = [0]

        def emit(origin, half, chunk):
            slot = n_emitted[0] % 2
            acc = jnp.dot(
                chunk, w8_ref[...], preferred_element_type=jnp.float32
            )
            if pending[slot] is not None:
                pending[slot].wait()
            stage_ref[slot] = jnp.maximum(acc * s_ref[0, 0], 0.0)
            row0 = origin * M_PER + half * M_HALF
            copy = pltpu.make_async_copy(
                stage_ref.at[slot],
                out_ref.at[pl.ds(row0, M_HALF), :],
                copy_sems.at[slot],
            )
            copy.start()
            pending[slot] = copy
            n_emitted[0] += 1

        emit(my_pos, 0, comm_r[0])
        emit(my_pos, 1, comm_l[0])

        for h in range(N_DEV - 1):
            d_r[h].wait_recv()
            if h + 1 < N_DEV - 1:
                d_r[h + 1].start()
            d_l[h].wait_recv()
            if h + 1 < N_DEV - 1:
                d_l[h + 1].start()
            origin_r = lax.rem(my_pos + N_DEV - 1 - h, N_DEV)
            origin_l = lax.rem(my_pos + h + 1, N_DEV)
            emit(origin_r, 0, comm_r[h + 1])
            emit(origin_l, 1, comm_l[h + 1])

        for h in range(N_DEV - 1):
            d_r[h].wait_send()
            d_l[h].wait_send()
        for p in pending:
            if p is not None:
                p.wait()

    return pl.pallas_call(
        body,
        out_shape=jax.ShapeDtypeStruct((N_DEV * M_PER, N_PER), jnp.float32),
        in_specs=[
            pl.BlockSpec(memory_space=pltpu.VMEM),
            pl.BlockSpec(memory_space=pl.ANY),
            pl.BlockSpec(memory_space=pltpu.SMEM),
            pl.BlockSpec(memory_space=pl.ANY),
        ],
        out_specs=pl.BlockSpec(memory_space=pl.ANY),
        input_output_aliases={3: 0},
        scratch_shapes=[
            pltpu.VMEM((N_DEV, M_HALF, K), jnp.float8_e4m3fn),
            pltpu.VMEM((N_DEV, M_HALF, K), jnp.float8_e4m3fn),
            pltpu.VMEM((K, N_PER), jnp.float8_e5m2),
            pltpu.VMEM((KT, N_PER), jnp.float32),
            pltpu.VMEM((2, M_HALF, N_PER), jnp.float32),
            pltpu.SemaphoreType.DMA((N_DEV - 1,)),
            pltpu.SemaphoreType.DMA((N_DEV - 1,)),
            pltpu.SemaphoreType.DMA((N_DEV - 1,)),
            pltpu.SemaphoreType.DMA((N_DEV - 1,)),
            pltpu.SemaphoreType.DMA((2,)),
            pltpu.SemaphoreType.DMA,
        ],
        compiler_params=pltpu.CompilerParams(
            collective_id=0, vmem_limit_bytes=100 * 1024 * 1024
        ),
    )(x, w_mat, scale, jnp.zeros((N_DEV * M_PER, N_PER), jnp.float32))


# device time: 110768 ns/iter; 1.1744x vs baseline; 1.1744x over previous
import jax
import jax.numpy as jnp
from jax import lax
from jax.experimental import pallas as pl
from jax.experimental.pallas import tpu as pltpu

N_DEV = 4
N_HOP = N_DEV - 1
M_PER = 1024
M_HALF = M_PER // 2
M_SUB = 256
Q = M_HALF // M_SUB
K = 4096
N_PER = 2048
KT = 1024
N_STAGE = 4


def kernel(x, w_mat, scale_x, scale_w):
    scale = jnp.reshape(scale_x[0] * scale_w[0], (1, 1))

    def body(
        x_ref, w_hbm, s_ref, out_ref, comm_r, comm_l, w8_ref, wtile_ref,
        stage_ref, send_r, recv_r, send_l, recv_l, copy_sems, w_sem,
    ):
        my_pos = lax.axis_index("i")
        left = lax.rem(my_pos + N_DEV - 1, N_DEV)
        right = lax.rem(my_pos + 1, N_DEV)

        barrier = pltpu.get_barrier_semaphore()
        for nbr in (left, right):
            pl.semaphore_signal(
                barrier,
                inc=1,
                device_id=(nbr,),
                device_id_type=pl.DeviceIdType.MESH,
            )
        pl.semaphore_wait(barrier, 2)

        def hop(comm, sems_s, sems_r, tgt, h, q):
            return pltpu.make_async_remote_copy(
                src_ref=comm.at[h, pl.ds(q * M_SUB, M_SUB)],
                dst_ref=comm.at[h + 1, pl.ds(q * M_SUB, M_SUB)],
                send_sem=sems_s.at[h, q],
                recv_sem=sems_r.at[h, q],
                device_id=(tgt,),
                device_id_type=pl.DeviceIdType.MESH,
            )

        d_r = [
            [hop(comm_r, send_r, recv_r, right, h, q) for q in range(Q)]
            for h in range(N_HOP)
        ]
        d_l = [
            [hop(comm_l, send_l, recv_l, left, h, q) for q in range(Q)]
            for h in range(N_HOP)
        ]

        for q in range(Q):
            comm_r[0, pl.ds(q * M_SUB, M_SUB)] = x_ref[
                pl.ds(q * M_SUB, M_SUB), :
            ].astype(jnp.float8_e4m3fn)
            d_r[0][q].start()
            comm_l[0, pl.ds(q * M_SUB, M_SUB)] = x_ref[
                pl.ds(M_HALF + q * M_SUB, M_SUB), :
            ].astype(jnp.float8_e4m3fn)
            d_l[0][q].start()

        col0 = my_pos * N_PER
        for kt in range(K // KT):
            wcopy = pltpu.make_async_copy(
                w_hbm.at[pl.ds(kt * KT, KT), pl.ds(col0, N_PER)],
                wtile_ref,
                w_sem,
            )
            wcopy.start()
            wcopy.wait()
            w8_ref[pl.ds(kt * KT, KT), :] = wtile_ref[...].astype(
                jnp.float8_e5m2
            )

        pending = [None] * N_STAGE
        n_emitted = [0]

        def emit(row0, chunk):
            slot = n_emitted[0] % N_STAGE
            acc = jnp.dot(
                chunk, w8_ref[...], preferred_element_type=jnp.float32
            )
            if pending[slot] is not None:
                pending[slot].wait()
            stage_ref[slot] = jnp.maximum(acc * s_ref[0, 0], 0.0)
            copy = pltpu.make_async_copy(
                stage_ref.at[slot],
                out_ref.at[pl.ds(row0, M_SUB), :],
                copy_sems.at[slot],
            )
            copy.start()
            pending[slot] = copy
            n_emitted[0] += 1

        for q in range(Q):
            emit(my_pos * M_PER + q * M_SUB, comm_r[0, pl.ds(q * M_SUB, M_SUB)])
            emit(
                my_pos * M_PER + M_HALF + q * M_SUB,
                comm_l[0, pl.ds(q * M_SUB, M_SUB)],
            )

        for h in range(N_HOP):
            origin_r = lax.rem(my_pos + N_DEV - 1 - h, N_DEV)
            origin_l = lax.rem(my_pos + h + 1, N_DEV)
            for q in range(Q):
                d_r[h][q].wait_recv()
                if h + 1 < N_HOP:
                    d_r[h + 1][q].start()
                emit(
                    origin_r * M_PER + q * M_SUB,
                    comm_r[h + 1, pl.ds(q * M_SUB, M_SUB)],
                )
                d_l[h][q].wait_recv()
                if h + 1 < N_HOP:
                    d_l[h + 1][q].start()
                emit(
                    origin_l * M_PER + M_HALF + q * M_SUB,
                    comm_l[h + 1, pl.ds(q * M_SUB, M_SUB)],
                )

        for h in range(N_HOP):
            for q in range(Q):
                d_r[h][q].wait_send()
                d_l[h][q].wait_send()
        for p in pending:
            if p is not None:
                p.wait()

    return pl.pallas_call(
        body,
        out_shape=jax.ShapeDtypeStruct((N_DEV * M_PER, N_PER), jnp.float32),
        in_specs=[
            pl.BlockSpec(memory_space=pltpu.VMEM),
            pl.BlockSpec(memory_space=pl.ANY),
            pl.BlockSpec(memory_space=pltpu.SMEM),
        ],
        out_specs=pl.BlockSpec(memory_space=pl.ANY),
        scratch_shapes=[
            pltpu.VMEM((N_DEV, M_HALF, K), jnp.float8_e4m3fn),
            pltpu.VMEM((N_DEV, M_HALF, K), jnp.float8_e4m3fn),
            pltpu.VMEM((K, N_PER), jnp.float8_e5m2),
            pltpu.VMEM((KT, N_PER), jnp.float32),
            pltpu.VMEM((N_STAGE, M_SUB, N_PER), jnp.float32),
            pltpu.SemaphoreType.DMA((N_HOP, Q)),
            pltpu.SemaphoreType.DMA((N_HOP, Q)),
            pltpu.SemaphoreType.DMA((N_HOP, Q)),
            pltpu.SemaphoreType.DMA((N_HOP, Q)),
            pltpu.SemaphoreType.DMA((N_STAGE,)),
            pltpu.SemaphoreType.DMA,
        ],
        compiler_params=pltpu.CompilerParams(
            collective_id=0, vmem_limit_bytes=100 * 1024 * 1024
        ),
    )(x, w_mat, scale)
